# baseline (speedup 1.0000x reference)
import jax
import jax.numpy as jnp
from jax import lax
from jax.experimental import pallas as pl
from jax.experimental.pallas import tpu as pltpu
from jax.experimental.pallas import tpu_sc as plsc

N = 10000; E = 160000; T = 8; H = 128; B = 16
NTILES = 32
PER_TILE = E // NTILES
CHUNK = 128
NCHUNK = 40
CAP = NCHUNK * CHUNK
N_ACC = 10240
RPT = N_ACC // 16
BN = 1000; NB = N // BN
BNC = 400; NBC = N // BNC

def _mesh():
    return plsc.VectorSubcoreMesh(core_axis_name="c", subcore_axis_name="s",
                                  num_cores=2, num_subcores=16)


def _sc_deg(dst_tbl, ones128, zeros128):
    def body(dst_hbm, ones_hbm, zer_hbm, out_hbm, dst_v, ones_v, zbuf, accum):
        c = lax.axis_index("c"); s = lax.axis_index("s")
        wid = c * 16 + s
        base = s * RPT
        pltpu.sync_copy(dst_hbm.at[wid], dst_v)
        pltpu.sync_copy(ones_hbm, ones_v)
        pltpu.sync_copy(zer_hbm, zbuf)
        for k in range(RPT // 128):
            pltpu.sync_copy(zbuf, accum.at[pl.ds(base + k * 128, 128)])
        plsc.subcore_barrier()

        def chunk(j, carry):
            pltpu.sync_copy(ones_v, accum.at[dst_v.at[j]], add=True)
            return carry
        lax.fori_loop(0, NCHUNK, chunk, 0)
        plsc.subcore_barrier()
        for k in range(RPT // 128):
            pltpu.sync_copy(accum.at[pl.ds(base + k * 128, 128)], ones_v)
            pltpu.sync_copy(ones_v, out_hbm.at[c, pl.ds(base + k * 128, 128)])

    f = pl.kernel(
        body,
        jax.ShapeDtypeStruct((2, N_ACC, H), jnp.float32),
        mesh=_mesh(),
        scratch_types=[
            pltpu.VMEM((NCHUNK, CHUNK), jnp.int32),
            pltpu.VMEM((CHUNK, H), jnp.float32),
            pltpu.VMEM((128, H), jnp.float32),
            pltpu.VMEM_SHARED((N_ACC, H), jnp.float32),
        ],
    )
    return f(dst_tbl, ones128, zeros128)


def _sc_agg(hp_flat, src_tbl, dst_tbl, zeros128):
    def body(hp_hbm, src_hbm, dst_hbm, zer_hbm, out_hbm,
             src_v, dst_v, stage_a, stage_b, rows_a, rows_b, accum,
             sem_a, sem_b, sem_sa, sem_sb):
        c = lax.axis_index("c"); s = lax.axis_index("s")
        wid = c * 16 + s
        base = s * RPT
        pltpu.sync_copy(src_hbm.at[wid], src_v)
        pltpu.sync_copy(dst_hbm.at[wid], dst_v)
        nc2 = NCHUNK // 2

        for t in range(T):
            for k in range(RPT // 128):
                pltpu.sync_copy(zer_hbm, accum.at[pl.ds(base + k * 128, 128)])
            plsc.subcore_barrier()
            toff = t * N

            def build(stage, j):
                for k in range(CHUNK // 16):
                    stage[pl.ds(k * 16, 16)] = (
                        src_v[pl.ds(j * CHUNK + k * 16, 16)] + toff)

            build(stage_a, 0)
            pltpu.async_copy(hp_hbm.at[stage_a], rows_a, sem_a)

            def pair(j2, carry):
                ja = 2 * j2
                jb = 2 * j2 + 1

                @pl.when(j2 > 0)
                def _():
                    pltpu.make_async_copy(
                        rows_b, accum.at[dst_v.at[jb]], sem_sb).wait()
                build(stage_b, jb)
                pltpu.async_copy(hp_hbm.at[stage_b], rows_b, sem_b)
                pltpu.make_async_copy(hp_hbm.at[stage_a], rows_a,
                                      sem_a).wait()
                pltpu.async_copy(rows_a, accum.at[dst_v.at[ja]], sem_sa,
                                 add=True)
                pltpu.make_async_copy(hp_hbm.at[stage_b], rows_b,
                                      sem_b).wait()
                pltpu.async_copy(rows_b, accum.at[dst_v.at[jb]], sem_sb,
                                 add=True)

                @pl.when(j2 + 1 < nc2)
                def _():
                    pltpu.make_async_copy(
                        rows_a, accum.at[dst_v.at[ja]], sem_sa).wait()
                    build(stage_a, ja + 2)
                    pltpu.async_copy(hp_hbm.at[stage_a], rows_a, sem_a)
                return carry
            lax.fori_loop(0, nc2, pair, 0)
            pltpu.make_async_copy(rows_a, accum.at[dst_v.at[0]],
                                  sem_sa).wait()
            pltpu.make_async_copy(rows_b, accum.at[dst_v.at[0]],
                                  sem_sb).wait()
            plsc.subcore_barrier()
            for k in range(RPT // 128):
                pltpu.sync_copy(accum.at[pl.ds(base + k * 128, 128)], rows_a)
                pltpu.sync_copy(
                    rows_a, out_hbm.at[c, t, pl.ds(base + k * 128, 128)])

    f = pl.kernel(
        body,
        jax.ShapeDtypeStruct((2, T, N_ACC, H), jnp.float32),
        mesh=_mesh(),
        scratch_types=[
            pltpu.VMEM((CAP,), jnp.int32),
            pltpu.VMEM((NCHUNK, CHUNK), jnp.int32),
            pltpu.VMEM((CHUNK,), jnp.int32),
            pltpu.VMEM((CHUNK,), jnp.int32),
            pltpu.VMEM((CHUNK, H), jnp.float32),
            pltpu.VMEM((CHUNK, H), jnp.float32),
            pltpu.VMEM_SHARED((N_ACC, H), jnp.float32),
            pltpu.SemaphoreType.DMA,
            pltpu.SemaphoreType.DMA,
            pltpu.SemaphoreType.DMA,
            pltpu.SemaphoreType.DMA,
        ],
    )
    return f(hp_flat, src_tbl, dst_tbl, zeros128)


def _ln(x, g, b):
    m = jnp.mean(x, axis=-1, keepdims=True)
    v = jnp.var(x, axis=-1, keepdims=True)
    return (x - m) * lax.rsqrt(v + 1e-5) * g + b


def _p_body(era5_ref, bcc_ref, bdc_ref, d0_ref, d1_ref, wce_ref, wcb_ref,
            bc_ref, wd_ref, bd_ref, w1_ref, b1_ref,
            proj_ref, hp_ref, dinv_ref):
    e = era5_ref[0]
    lin = (jnp.dot(e, wce_ref[...], preferred_element_type=jnp.float32)
           + jnp.dot(bcc_ref[...], wcb_ref[...],
                     preferred_element_type=jnp.float32) + bc_ref[...])
    gate = jax.nn.sigmoid(
        jnp.dot(bdc_ref[...], wd_ref[...],
                preferred_element_type=jnp.float32) + bd_ref[...])
    proj = jnp.tanh(lin) * gate
    proj_ref[0] = proj
    dinv = lax.rsqrt(d0_ref[0][:, :1] + d1_ref[0][:, :1] + 1.0)
    dinv_ref[...] = dinv
    h1 = jnp.dot(proj, w1_ref[...],
                 preferred_element_type=jnp.float32) + b1_ref[...]
    hp_ref[0] = dinv * h1


def _proj_kernel(era5_t, bcc, bdc, degp, bp, blk0):
    return pl.pallas_call(
        _p_body,
        grid=(T, NB),
        in_specs=[
            pl.BlockSpec((1, BN, 16), lambda t, i: (t, i, 0)),
            pl.BlockSpec((BN, 32), lambda t, i: (i, 0)),
            pl.BlockSpec((BN, 8), lambda t, i: (i, 0)),
            pl.BlockSpec((1, BN, H), lambda t, i: (0, i, 0)),
            pl.BlockSpec((1, BN, H), lambda t, i: (1, i, 0)),
            pl.BlockSpec((16, H), lambda t, i: (0, 0)),
            pl.BlockSpec((32, H), lambda t, i: (0, 0)),
            pl.BlockSpec((H,), lambda t, i: (0,)),
            pl.BlockSpec((8, H), lambda t, i: (0, 0)),
            pl.BlockSpec((H,), lambda t, i: (0,)),
            pl.BlockSpec((H, H), lambda t, i: (0, 0)),
            pl.BlockSpec((H,), lambda t, i: (0,)),
        ],
        out_specs=[
            pl.BlockSpec((1, BN, H), lambda t, i: (t, i, 0)),
            pl.BlockSpec((1, BN, H), lambda t, i: (t, i, 0)),
            pl.BlockSpec((BN, 1), lambda t, i: (i, 0)),
        ],
        out_shape=[
            jax.ShapeDtypeStruct((T, N, H), jnp.float32),
            jax.ShapeDtypeStruct((T, N, H), jnp.float32),
            jax.ShapeDtypeStruct((N, 1), jnp.float32),
        ],
    )(era5_t, bcc, bdc, degp, degp, bp['Wc'][:16], bp['Wc'][16:], bp['bc'],
      bp['Wd'], bp['bd'], blk0['gcn_W1'], blk0['gcn_b1'])


def _b_body(p_ref, hp_ref, dinv_ref, w2_ref, b2_ref, out_ref):
    dinv = dinv_ref[...]
    agg = dinv * (p_ref[0, 0] + p_ref[1, 0] + hp_ref[0])
    h = jnp.maximum(agg, 0.0)
    h2 = jnp.dot(h, w2_ref[...],
                 preferred_element_type=jnp.float32) + b2_ref[...]
    out_ref[0] = dinv * h2


def _b_kernel(P, hp, dinv, w2, b2):
    return pl.pallas_call(
        _b_body,
        grid=(T, NB),
        in_specs=[
            pl.BlockSpec((2, 1, BN, H), lambda t, i: (0, t, i, 0)),
            pl.BlockSpec((1, BN, H), lambda t, i: (t, i, 0)),
            pl.BlockSpec((BN, 1), lambda t, i: (i, 0)),
            pl.BlockSpec((H, H), lambda t, i: (0, 0)),
            pl.BlockSpec((H,), lambda t, i: (0,)),
        ],
        out_specs=pl.BlockSpec((1, BN, H), lambda t, i: (t, i, 0)),
        out_shape=jax.ShapeDtypeStruct((T, N, H), jnp.float32),
    )(P, hp, dinv, w2, b2)


def _c_body_mid(p_ref, hp_ref, dinv_ref, proj_ref, ln2g_ref, ln2b_ref,
                wih_ref, whh_ref, bih_ref, bhh_ref, ln1g_ref, ln1b_ref,
                w1n_ref, b1n_ref, proj_out_ref, hpn_out_ref):
    dinv = dinv_ref[...]
    bsum = bih_ref[...] + bhh_ref[...]
    wih = wih_ref[...]
    whh = whh_ref[...]
    hh = jnp.zeros((BNC, H), jnp.float32)
    cc = jnp.zeros((BNC, H), jnp.float32)
    for t in range(T):
        g2 = dinv * (p_ref[0, t] + p_ref[1, t] + hp_ref[t])
        x = _ln(g2, ln2g_ref[...], ln2b_ref[...])
        gates = (jnp.dot(x, wih, preferred_element_type=jnp.float32)
                 + jnp.dot(hh, whh, preferred_element_type=jnp.float32)
                 + bsum)
        ig = jax.nn.sigmoid(gates[:, 0:H])
        fg = jax.nn.sigmoid(gates[:, H:2 * H])
        gg = jnp.tanh(gates[:, 2 * H:3 * H])
        og = jax.nn.sigmoid(gates[:, 3 * H:4 * H])
        cc = fg * cc + ig * gg
        hh = og * jnp.tanh(cc)
        st = _ln(hh, ln1g_ref[...], ln1b_ref[...])
        newp = proj_ref[t] + st
        proj_out_ref[t] = newp
        hpn_out_ref[t] = dinv * (
            jnp.dot(newp, w1n_ref[...],
                    preferred_element_type=jnp.float32) + b1n_ref[...])


def _c_body_final(p_ref, hp_ref, dinv_ref, proj_ref, ln2g_ref, ln2b_ref,
                  wih_ref, whh_ref, bih_ref, bhh_ref, ln1g_ref, ln1b_ref,
                  hbw_ref, hbb_ref, cbw_ref, cbb_ref,
                  proj_out_ref, hh_out_ref, cc_out_ref):
    dinv = dinv_ref[...]
    bsum = bih_ref[...] + bhh_ref[...]
    wih = wih_ref[...]
    whh = whh_ref[...]
    hh = jnp.zeros((BNC, H), jnp.float32)
    cc = jnp.zeros((BNC, H), jnp.float32)
    for t in range(T):
        g2 = dinv * (p_ref[0, t] + p_ref[1, t] + hp_ref[t])
        x = _ln(g2, ln2g_ref[...], ln2b_ref[...])
        gates = (jnp.dot(x, wih, preferred_element_type=jnp.float32)
                 + jnp.dot(hh, whh, preferred_element_type=jnp.float32)
                 + bsum)
        ig = jax.nn.sigmoid(gates[:, 0:H])
        fg = jax.nn.sigmoid(gates[:, H:2 * H])
        gg = jnp.tanh(gates[:, 2 * H:3 * H])
        og = jax.nn.sigmoid(gates[:, 3 * H:4 * H])
        cc = fg * cc + ig * gg
        hh = og * jnp.tanh(cc)
        st = _ln(hh, ln1g_ref[...], ln1b_ref[...])
        proj_out_ref[:, t, :] = proj_ref[t] + st
    hh_out_ref[...] = jnp.tanh(
        jnp.dot(hh, hbw_ref[...],
                preferred_element_type=jnp.float32) + hbb_ref[...])
    cc_out_ref[...] = (jnp.dot(cc, cbw_ref[...],
                               preferred_element_type=jnp.float32)
                       + cbb_ref[...])


_C_COMMON_SPECS = [
    pl.BlockSpec((2, T, BNC, H), lambda i: (0, 0, i, 0)),
    pl.BlockSpec((T, BNC, H), lambda i: (0, i, 0)),
    pl.BlockSpec((BNC, 1), lambda i: (i, 0)),
    pl.BlockSpec((T, BNC, H), lambda i: (0, i, 0)),
    pl.BlockSpec((H,), lambda i: (0,)),
    pl.BlockSpec((H,), lambda i: (0,)),
    pl.BlockSpec((H, 4 * H), lambda i: (0, 0)),
    pl.BlockSpec((H, 4 * H), lambda i: (0, 0)),
    pl.BlockSpec((4 * H,), lambda i: (0,)),
    pl.BlockSpec((4 * H,), lambda i: (0,)),
    pl.BlockSpec((H,), lambda i: (0,)),
    pl.BlockSpec((H,), lambda i: (0,)),
]


def _c_kernel_mid(P, hp, dinv, proj, bp, nextp):
    return pl.pallas_call(
        _c_body_mid,
        grid=(NBC,),
        in_specs=_C_COMMON_SPECS + [
            pl.BlockSpec((H, H), lambda i: (0, 0)),
            pl.BlockSpec((H,), lambda i: (0,)),
        ],
        out_specs=[
            pl.BlockSpec((T, BNC, H), lambda i: (0, i, 0)),
            pl.BlockSpec((T, BNC, H), lambda i: (0, i, 0)),
        ],
        out_shape=[
            jax.ShapeDtypeStruct((T, N, H), jnp.float32),
            jax.ShapeDtypeStruct((T, N, H), jnp.float32),
        ],
    )(P, hp, dinv, proj, bp['ln2_g'], bp['ln2_b'], bp['Wih'].T, bp['Whh'].T,
      bp['bih'], bp['bhh'], bp['ln1_g'], bp['ln1_b'],
      nextp['gcn_W1'], nextp['gcn_b1'])


def _c_kernel_final(P, hp, dinv, proj, bp):
    return pl.pallas_call(
        _c_body_final,
        grid=(NBC,),
        in_specs=_C_COMMON_SPECS + [
            pl.BlockSpec((H, H), lambda i: (0, 0)),
            pl.BlockSpec((H,), lambda i: (0,)),
            pl.BlockSpec((H, H), lambda i: (0, 0)),
            pl.BlockSpec((H,), lambda i: (0,)),
        ],
        out_specs=[
            pl.BlockSpec((BNC, T, H), lambda i: (i, 0, 0)),
            pl.BlockSpec((BNC, H), lambda i: (i, 0)),
            pl.BlockSpec((BNC, H), lambda i: (i, 0)),
        ],
        out_shape=[
            jax.ShapeDtypeStruct((N, T, H), jnp.float32),
            jax.ShapeDtypeStruct((N, H), jnp.float32),
            jax.ShapeDtypeStruct((N, H), jnp.float32),
        ],
    )(P, hp, dinv, proj, bp['ln2_g'], bp['ln2_b'], bp['Wih'].T, bp['Whh'].T,
      bp['bih'], bp['bhh'], bp['ln1_g'], bp['ln1_b'],
      bp['hb_W'], bp['hb_b'], bp['cb_W'], bp['cb_b'])


def _idx_body(nodes_ref, out_ref):
    x = nodes_ref[...].astype(jnp.float32)
    r = lax.broadcasted_iota(jnp.int32, (16, 16), 0)
    c = lax.broadcasted_iota(jnp.int32, (16, 16), 1)
    tri = (r < c).astype(jnp.float32)
    out_ref[...] = jnp.dot(
        x, tri, preferred_element_type=jnp.float32,
        precision=lax.Precision.HIGHEST).astype(jnp.int32)


def _idx_kernel(nodes):
    return pl.pallas_call(
        _idx_body,
        out_shape=jax.ShapeDtypeStruct((1, 16), jnp.int32),
    )(nodes.reshape(1, 16))


def _d_body(idx_ref, proj_ref, rc_ref, rd_ref, wcs_ref, wcr_ref, bc_ref,
            wd_ref, bd_ref, wh_ref, bh_ref, out_ref):
    del idx_ref
    b = pl.program_id(0)
    s = proj_ref[0]
    rc = rc_ref[pl.ds(b, 1), :]
    rd = rd_ref[pl.ds(b, 1), :]
    lin = (jnp.dot(s, wcs_ref[...], preferred_element_type=jnp.float32)
           + jnp.dot(rc, wcr_ref[...],
                     preferred_element_type=jnp.float32) + bc_ref[...])
    gate = jax.nn.sigmoid(
        jnp.dot(rd, wd_ref[...],
                preferred_element_type=jnp.float32) + bd_ref[...])
    river = jnp.tanh(lin) * gate
    raw = jnp.dot(river, wh_ref[...],
                  preferred_element_type=jnp.float32) + bh_ref[...]
    col = lax.broadcasted_iota(jnp.int32, (T, 12), 1)
    pim = col >= 9
    m = jnp.max(jnp.where(pim, raw, -1e30), axis=1, keepdims=True)
    e = jnp.exp(jnp.where(pim, raw - m, -30.0))
    pi = e / jnp.sum(jnp.where(pim, e, 0.0), axis=1, keepdims=True)
    val = jnp.where(col < 3, raw,
                    jnp.where(col < 6, jax.nn.softplus(raw),
                              jnp.where(col < 9, jax.nn.sigmoid(raw), pi)))
    out_ref[0] = val


def _d_kernel(batch_idx, proj_final, rc, rd, rp, head):
    wh = jnp.concatenate(
        [head['W_mu'], head['W_b'], head['W_tau'], head['W_pi']], axis=1)
    bh = jnp.concatenate(
        [head['b_mu'], head['b_b'], head['b_tau'], head['b_pi']], axis=0)
    grid_spec = pltpu.PrefetchScalarGridSpec(
        num_scalar_prefetch=1,
        grid=(B,),
        in_specs=[
            pl.BlockSpec((1, T, H), lambda b, idx: (idx[b], 0, 0)),
            pl.BlockSpec((B, 32), lambda b, idx: (0, 0)),
            pl.BlockSpec((B, 8), lambda b, idx: (0, 0)),
            pl.BlockSpec((H, H), lambda b, idx: (0, 0)),
            pl.BlockSpec((32, H), lambda b, idx: (0, 0)),
            pl.BlockSpec((H,), lambda b, idx: (0,)),
            pl.BlockSpec((8, H), lambda b, idx: (0, 0)),
            pl.BlockSpec((H,), lambda b, idx: (0,)),
            pl.BlockSpec((H, 12), lambda b, idx: (0, 0)),
            pl.BlockSpec((12,), lambda b, idx: (0,)),
        ],
        out_specs=pl.BlockSpec((1, T, 12), lambda b, idx: (b, 0, 0)),
    )
    return pl.pallas_call(
        _d_body,
        grid_spec=grid_spec,
        out_shape=jax.ShapeDtypeStruct((B, T, 12), jnp.float32),
    )(batch_idx, proj_final, rc, rd, rp['Wc'][:H], rp['Wc'][H:], rp['bc'],
      rp['Wd'], rp['bd'], wh, bh)


def kernel(era5, basinContinuous, basinDiscrete, riverContinuous,
           riverDiscrete, edge_index, nodes, params):
    src = edge_index[0]
    dst = edge_index[1]
    src_tbl = jnp.pad(src.reshape(NTILES, PER_TILE),
                      ((0, 0), (0, CAP - PER_TILE)))
    dst_tbl = jnp.pad(dst.reshape(NTILES, PER_TILE),
                      ((0, 0), (0, CAP - PER_TILE)),
                      constant_values=N).reshape(NTILES, NCHUNK, CHUNK)
    zeros128 = jnp.zeros((128, H), jnp.float32)
    ones128 = jnp.ones((CHUNK, H), jnp.float32)
    era5_t = jnp.transpose(era5, (1, 0, 2))

    degp3 = _sc_deg(dst_tbl, ones128, zeros128)

    blocks = params['blocks']
    proj, hp, dinv = _proj_kernel(era5_t, basinContinuous, basinDiscrete,
                                  degp3, params['basinProj'], blocks[0])

    Pa = _sc_agg(hp.reshape(T * N, H), src_tbl, dst_tbl, zeros128)
    hp2 = _b_kernel(Pa, hp, dinv, blocks[0]['gcn_W2'], blocks[0]['gcn_b2'])
    Pb = _sc_agg(hp2.reshape(T * N, H), src_tbl, dst_tbl, zeros128)
    proj1, hp_b1 = _c_kernel_mid(Pb, hp2, dinv, proj, blocks[0], blocks[1])

    Pc = _sc_agg(hp_b1.reshape(T * N, H), src_tbl, dst_tbl, zeros128)
    hp2_b1 = _b_kernel(Pc, hp_b1, dinv,
                       blocks[1]['gcn_W2'], blocks[1]['gcn_b2'])
    Pd = _sc_agg(hp2_b1.reshape(T * N, H), src_tbl, dst_tbl, zeros128)
    proj_final, hh, cc = _c_kernel_final(Pd, hp2_b1, dinv, proj1, blocks[1])

    batch_idx = _idx_kernel(nodes).reshape(B)
    cast = _d_kernel(batch_idx, proj_final, riverContinuous, riverDiscrete,
                     params['riverProj'], params['head'])
    return cast, hh, cc


# speedup vs baseline: 1.1196x; 1.1196x over previous
import jax
import jax.numpy as jnp
from jax import lax
from jax.experimental import pallas as pl
from jax.experimental.pallas import tpu as pltpu
from jax.experimental.pallas import tpu_sc as plsc

N = 10000; E = 160000; T = 8; H = 128; B = 16
NTILES = 32
PER_TILE = E // NTILES
CHUNK = 128
NCHUNK = 40
CAP = NCHUNK * CHUNK
N_ACC = 10240
RPT = N_ACC // 16
BN = 1000; NB = N // BN
BNC = 400; NBC = N // BNC

def _mesh():
    return plsc.VectorSubcoreMesh(core_axis_name="c", subcore_axis_name="s",
                                  num_cores=2, num_subcores=16)


def _sc_deg(dst_tbl, ones128, zeros128):
    def body(dst_hbm, ones_hbm, zer_hbm, out_hbm, dst_v, ones_v, zbuf, accum):
        c = lax.axis_index("c"); s = lax.axis_index("s")
        wid = c * 16 + s
        base = s * RPT
        pltpu.sync_copy(dst_hbm.at[wid], dst_v)
        pltpu.sync_copy(ones_hbm, ones_v)
        pltpu.sync_copy(zer_hbm, zbuf)
        for k in range(RPT // 128):
            pltpu.sync_copy(zbuf, accum.at[pl.ds(base + k * 128, 128)])
        plsc.subcore_barrier()

        def chunk(j, carry):
            pltpu.sync_copy(ones_v, accum.at[dst_v.at[j]], add=True)
            return carry
        lax.fori_loop(0, NCHUNK, chunk, 0)
        plsc.subcore_barrier()
        for k in range(RPT // 128):
            pltpu.sync_copy(accum.at[pl.ds(base + k * 128, 128)], ones_v)
            pltpu.sync_copy(ones_v, out_hbm.at[c, pl.ds(base + k * 128, 128)])

    f = pl.kernel(
        body,
        jax.ShapeDtypeStruct((2, N_ACC, H), jnp.float32),
        mesh=_mesh(),
        scratch_types=[
            pltpu.VMEM((NCHUNK, CHUNK), jnp.int32),
            pltpu.VMEM((CHUNK, H), jnp.float32),
            pltpu.VMEM((128, H), jnp.float32),
            pltpu.VMEM_SHARED((N_ACC, H), jnp.float32),
        ],
    )
    return f(dst_tbl, ones128, zeros128)


def _sc_agg(hp_flat, src_tbl, dst_tbl, zeros640):
    def body(hp_hbm, src_hbm, dst_hbm, zer_hbm, out_hbm,
             src_v, dst_v, stage_a, stage_b, rows_a, rows_b, accum,
             sem_a, sem_b):
        c = lax.axis_index("c"); s = lax.axis_index("s")
        wid = c * 16 + s
        base = s * RPT
        pltpu.sync_copy(src_hbm.at[wid], src_v)
        pltpu.sync_copy(dst_hbm.at[wid], dst_v)
        nc2 = NCHUNK // 2

        for t in range(T):
            pltpu.sync_copy(zer_hbm, accum.at[pl.ds(base, RPT)])
            plsc.subcore_barrier()
            toff = t * N

            def build(stage, j):
                for k in range(CHUNK // 16):
                    stage[pl.ds(k * 16, 16)] = (
                        src_v[pl.ds(j * CHUNK + k * 16, 16)] + toff)

            build(stage_a, 0)
            pltpu.async_copy(hp_hbm.at[stage_a], rows_a, sem_a)

            def pair(j2, carry):
                ja = 2 * j2
                jb = 2 * j2 + 1
                build(stage_b, jb)
                pltpu.async_copy(hp_hbm.at[stage_b], rows_b, sem_b)
                pltpu.make_async_copy(hp_hbm.at[stage_a], rows_a,
                                      sem_a).wait()
                pltpu.sync_copy(rows_a, accum.at[dst_v.at[ja]], add=True)

                @pl.when(j2 + 1 < nc2)
                def _():
                    build(stage_a, ja + 2)
                    pltpu.async_copy(hp_hbm.at[stage_a], rows_a, sem_a)
                pltpu.make_async_copy(hp_hbm.at[stage_b], rows_b,
                                      sem_b).wait()
                pltpu.sync_copy(rows_b, accum.at[dst_v.at[jb]], add=True)
                return carry
            lax.fori_loop(0, nc2, pair, 0)
            plsc.subcore_barrier()
            pltpu.sync_copy(accum.at[pl.ds(base, RPT)],
                            out_hbm.at[c, t, pl.ds(base, RPT)])

    f = pl.kernel(
        body,
        jax.ShapeDtypeStruct((2, T, N_ACC, H), jnp.float32),
        mesh=_mesh(),
        scratch_types=[
            pltpu.VMEM((CAP,), jnp.int32),
            pltpu.VMEM((NCHUNK, CHUNK), jnp.int32),
            pltpu.VMEM((CHUNK,), jnp.int32),
            pltpu.VMEM((CHUNK,), jnp.int32),
            pltpu.VMEM((CHUNK, H), jnp.float32),
            pltpu.VMEM((CHUNK, H), jnp.float32),
            pltpu.VMEM_SHARED((N_ACC, H), jnp.float32),
            pltpu.SemaphoreType.DMA,
            pltpu.SemaphoreType.DMA,
        ],
    )
    return f(hp_flat, src_tbl, dst_tbl, zeros640)


def _ln(x, g, b):
    m = jnp.mean(x, axis=-1, keepdims=True)
    v = jnp.var(x, axis=-1, keepdims=True)
    return (x - m) * lax.rsqrt(v + 1e-5) * g + b


def _p_body(era5_ref, bcc_ref, bdc_ref, d0_ref, d1_ref, wce_ref, wcb_ref,
            bc_ref, wd_ref, bd_ref, w1_ref, b1_ref,
            proj_ref, hp_ref, dinv_ref):
    e = era5_ref[0]
    lin = (jnp.dot(e, wce_ref[...], preferred_element_type=jnp.float32)
           + jnp.dot(bcc_ref[...], wcb_ref[...],
                     preferred_element_type=jnp.float32) + bc_ref[...])
    gate = jax.nn.sigmoid(
        jnp.dot(bdc_ref[...], wd_ref[...],
                preferred_element_type=jnp.float32) + bd_ref[...])
    proj = jnp.tanh(lin) * gate
    proj_ref[0] = proj
    dinv = lax.rsqrt(d0_ref[0][:, :1] + d1_ref[0][:, :1] + 1.0)
    dinv_ref[...] = dinv
    h1 = jnp.dot(proj, w1_ref[...],
                 preferred_element_type=jnp.float32) + b1_ref[...]
    hp_ref[0] = dinv * h1


def _proj_kernel(era5_t, bcc, bdc, degp, bp, blk0):
    return pl.pallas_call(
        _p_body,
        grid=(T, NB),
        in_specs=[
            pl.BlockSpec((1, BN, 16), lambda t, i: (t, i, 0)),
            pl.BlockSpec((BN, 32), lambda t, i: (i, 0)),
            pl.BlockSpec((BN, 8), lambda t, i: (i, 0)),
            pl.BlockSpec((1, BN, H), lambda t, i: (0, i, 0)),
            pl.BlockSpec((1, BN, H), lambda t, i: (1, i, 0)),
            pl.BlockSpec((16, H), lambda t, i: (0, 0)),
            pl.BlockSpec((32, H), lambda t, i: (0, 0)),
            pl.BlockSpec((H,), lambda t, i: (0,)),
            pl.BlockSpec((8, H), lambda t, i: (0, 0)),
            pl.BlockSpec((H,), lambda t, i: (0,)),
            pl.BlockSpec((H, H), lambda t, i: (0, 0)),
            pl.BlockSpec((H,), lambda t, i: (0,)),
        ],
        out_specs=[
            pl.BlockSpec((1, BN, H), lambda t, i: (t, i, 0)),
            pl.BlockSpec((1, BN, H), lambda t, i: (t, i, 0)),
            pl.BlockSpec((BN, 1), lambda t, i: (i, 0)),
        ],
        out_shape=[
            jax.ShapeDtypeStruct((T, N, H), jnp.float32),
            jax.ShapeDtypeStruct((T, N, H), jnp.float32),
            jax.ShapeDtypeStruct((N, 1), jnp.float32),
        ],
    )(era5_t, bcc, bdc, degp, degp, bp['Wc'][:16], bp['Wc'][16:], bp['bc'],
      bp['Wd'], bp['bd'], blk0['gcn_W1'], blk0['gcn_b1'])


def _b_body(p_ref, hp_ref, dinv_ref, w2_ref, b2_ref, out_ref):
    dinv = dinv_ref[...]
    agg = dinv * (p_ref[0, 0] + p_ref[1, 0] + hp_ref[0])
    h = jnp.maximum(agg, 0.0)
    h2 = jnp.dot(h, w2_ref[...],
                 preferred_element_type=jnp.float32) + b2_ref[...]
    out_ref[0] = dinv * h2


def _b_kernel(P, hp, dinv, w2, b2):
    return pl.pallas_call(
        _b_body,
        grid=(T, NB),
        in_specs=[
            pl.BlockSpec((2, 1, BN, H), lambda t, i: (0, t, i, 0)),
            pl.BlockSpec((1, BN, H), lambda t, i: (t, i, 0)),
            pl.BlockSpec((BN, 1), lambda t, i: (i, 0)),
            pl.BlockSpec((H, H), lambda t, i: (0, 0)),
            pl.BlockSpec((H,), lambda t, i: (0,)),
        ],
        out_specs=pl.BlockSpec((1, BN, H), lambda t, i: (t, i, 0)),
        out_shape=jax.ShapeDtypeStruct((T, N, H), jnp.float32),
    )(P, hp, dinv, w2, b2)


def _c_body_mid(p_ref, hp_ref, dinv_ref, proj_ref, ln2g_ref, ln2b_ref,
                wih_ref, whh_ref, bih_ref, bhh_ref, ln1g_ref, ln1b_ref,
                w1n_ref, b1n_ref, proj_out_ref, hpn_out_ref):
    dinv = dinv_ref[...]
    bsum = bih_ref[...] + bhh_ref[...]
    wih = wih_ref[...]
    whh = whh_ref[...]
    hh = jnp.zeros((BNC, H), jnp.float32)
    cc = jnp.zeros((BNC, H), jnp.float32)
    for t in range(T):
        g2 = dinv * (p_ref[0, t] + p_ref[1, t] + hp_ref[t])
        x = _ln(g2, ln2g_ref[...], ln2b_ref[...])
        gates = (jnp.dot(x, wih, preferred_element_type=jnp.float32)
                 + jnp.dot(hh, whh, preferred_element_type=jnp.float32)
                 + bsum)
        ig = jax.nn.sigmoid(gates[:, 0:H])
        fg = jax.nn.sigmoid(gates[:, H:2 * H])
        gg = jnp.tanh(gates[:, 2 * H:3 * H])
        og = jax.nn.sigmoid(gates[:, 3 * H:4 * H])
        cc = fg * cc + ig * gg
        hh = og * jnp.tanh(cc)
        st = _ln(hh, ln1g_ref[...], ln1b_ref[...])
        newp = proj_ref[t] + st
        proj_out_ref[t] = newp
        hpn_out_ref[t] = dinv * (
            jnp.dot(newp, w1n_ref[...],
                    preferred_element_type=jnp.float32) + b1n_ref[...])


def _c_body_final(p_ref, hp_ref, dinv_ref, proj_ref, ln2g_ref, ln2b_ref,
                  wih_ref, whh_ref, bih_ref, bhh_ref, ln1g_ref, ln1b_ref,
                  hbw_ref, hbb_ref, cbw_ref, cbb_ref,
                  proj_out_ref, hh_out_ref, cc_out_ref):
    dinv = dinv_ref[...]
    bsum = bih_ref[...] + bhh_ref[...]
    wih = wih_ref[...]
    whh = whh_ref[...]
    hh = jnp.zeros((BNC, H), jnp.float32)
    cc = jnp.zeros((BNC, H), jnp.float32)
    for t in range(T):
        g2 = dinv * (p_ref[0, t] + p_ref[1, t] + hp_ref[t])
        x = _ln(g2, ln2g_ref[...], ln2b_ref[...])
        gates = (jnp.dot(x, wih, preferred_element_type=jnp.float32)
                 + jnp.dot(hh, whh, preferred_element_type=jnp.float32)
                 + bsum)
        ig = jax.nn.sigmoid(gates[:, 0:H])
        fg = jax.nn.sigmoid(gates[:, H:2 * H])
        gg = jnp.tanh(gates[:, 2 * H:3 * H])
        og = jax.nn.sigmoid(gates[:, 3 * H:4 * H])
        cc = fg * cc + ig * gg
        hh = og * jnp.tanh(cc)
        st = _ln(hh, ln1g_ref[...], ln1b_ref[...])
        proj_out_ref[:, t, :] = proj_ref[t] + st
    hh_out_ref[...] = jnp.tanh(
        jnp.dot(hh, hbw_ref[...],
                preferred_element_type=jnp.float32) + hbb_ref[...])
    cc_out_ref[...] = (jnp.dot(cc, cbw_ref[...],
                               preferred_element_type=jnp.float32)
                       + cbb_ref[...])


_C_COMMON_SPECS = [
    pl.BlockSpec((2, T, BNC, H), lambda i: (0, 0, i, 0)),
    pl.BlockSpec((T, BNC, H), lambda i: (0, i, 0)),
    pl.BlockSpec((BNC, 1), lambda i: (i, 0)),
    pl.BlockSpec((T, BNC, H), lambda i: (0, i, 0)),
    pl.BlockSpec((H,), lambda i: (0,)),
    pl.BlockSpec((H,), lambda i: (0,)),
    pl.BlockSpec((H, 4 * H), lambda i: (0, 0)),
    pl.BlockSpec((H, 4 * H), lambda i: (0, 0)),
    pl.BlockSpec((4 * H,), lambda i: (0,)),
    pl.BlockSpec((4 * H,), lambda i: (0,)),
    pl.BlockSpec((H,), lambda i: (0,)),
    pl.BlockSpec((H,), lambda i: (0,)),
]


def _c_kernel_mid(P, hp, dinv, proj, bp, nextp):
    return pl.pallas_call(
        _c_body_mid,
        grid=(NBC,),
        in_specs=_C_COMMON_SPECS + [
            pl.BlockSpec((H, H), lambda i: (0, 0)),
            pl.BlockSpec((H,), lambda i: (0,)),
        ],
        out_specs=[
            pl.BlockSpec((T, BNC, H), lambda i: (0, i, 0)),
            pl.BlockSpec((T, BNC, H), lambda i: (0, i, 0)),
        ],
        out_shape=[
            jax.ShapeDtypeStruct((T, N, H), jnp.float32),
            jax.ShapeDtypeStruct((T, N, H), jnp.float32),
        ],
    )(P, hp, dinv, proj, bp['ln2_g'], bp['ln2_b'], bp['Wih'].T, bp['Whh'].T,
      bp['bih'], bp['bhh'], bp['ln1_g'], bp['ln1_b'],
      nextp['gcn_W1'], nextp['gcn_b1'])


def _c_kernel_final(P, hp, dinv, proj, bp):
    return pl.pallas_call(
        _c_body_final,
        grid=(NBC,),
        in_specs=_C_COMMON_SPECS + [
            pl.BlockSpec((H, H), lambda i: (0, 0)),
            pl.BlockSpec((H,), lambda i: (0,)),
            pl.BlockSpec((H, H), lambda i: (0, 0)),
            pl.BlockSpec((H,), lambda i: (0,)),
        ],
        out_specs=[
            pl.BlockSpec((BNC, T, H), lambda i: (i, 0, 0)),
            pl.BlockSpec((BNC, H), lambda i: (i, 0)),
            pl.BlockSpec((BNC, H), lambda i: (i, 0)),
        ],
        out_shape=[
            jax.ShapeDtypeStruct((N, T, H), jnp.float32),
            jax.ShapeDtypeStruct((N, H), jnp.float32),
            jax.ShapeDtypeStruct((N, H), jnp.float32),
        ],
    )(P, hp, dinv, proj, bp['ln2_g'], bp['ln2_b'], bp['Wih'].T, bp['Whh'].T,
      bp['bih'], bp['bhh'], bp['ln1_g'], bp['ln1_b'],
      bp['hb_W'], bp['hb_b'], bp['cb_W'], bp['cb_b'])


def _idx_body(nodes_ref, out_ref):
    x = nodes_ref[...].astype(jnp.float32)
    r = lax.broadcasted_iota(jnp.int32, (16, 16), 0)
    c = lax.broadcasted_iota(jnp.int32, (16, 16), 1)
    tri = (r < c).astype(jnp.float32)
    out_ref[...] = jnp.dot(
        x, tri, preferred_element_type=jnp.float32,
        precision=lax.Precision.HIGHEST).astype(jnp.int32)


def _idx_kernel(nodes):
    return pl.pallas_call(
        _idx_body,
        out_shape=jax.ShapeDtypeStruct((1, 16), jnp.int32),
    )(nodes.reshape(1, 16))


def _d_body(idx_ref, proj_ref, rc_ref, rd_ref, wcs_ref, wcr_ref, bc_ref,
            wd_ref, bd_ref, wh_ref, bh_ref, out_ref):
    del idx_ref
    b = pl.program_id(0)
    s = proj_ref[0]
    rc = rc_ref[pl.ds(b, 1), :]
    rd = rd_ref[pl.ds(b, 1), :]
    lin = (jnp.dot(s, wcs_ref[...], preferred_element_type=jnp.float32)
           + jnp.dot(rc, wcr_ref[...],
                     preferred_element_type=jnp.float32) + bc_ref[...])
    gate = jax.nn.sigmoid(
        jnp.dot(rd, wd_ref[...],
                preferred_element_type=jnp.float32) + bd_ref[...])
    river = jnp.tanh(lin) * gate
    raw = jnp.dot(river, wh_ref[...],
                  preferred_element_type=jnp.float32) + bh_ref[...]
    col = lax.broadcasted_iota(jnp.int32, (T, 12), 1)
    pim = col >= 9
    m = jnp.max(jnp.where(pim, raw, -1e30), axis=1, keepdims=True)
    e = jnp.exp(jnp.where(pim, raw - m, -30.0))
    pi = e / jnp.sum(jnp.where(pim, e, 0.0), axis=1, keepdims=True)
    val = jnp.where(col < 3, raw,
                    jnp.where(col < 6, jax.nn.softplus(raw),
                              jnp.where(col < 9, jax.nn.sigmoid(raw), pi)))
    out_ref[0] = val


def _d_kernel(batch_idx, proj_final, rc, rd, rp, head):
    wh = jnp.concatenate(
        [head['W_mu'], head['W_b'], head['W_tau'], head['W_pi']], axis=1)
    bh = jnp.concatenate(
        [head['b_mu'], head['b_b'], head['b_tau'], head['b_pi']], axis=0)
    grid_spec = pltpu.PrefetchScalarGridSpec(
        num_scalar_prefetch=1,
        grid=(B,),
        in_specs=[
            pl.BlockSpec((1, T, H), lambda b, idx: (idx[b], 0, 0)),
            pl.BlockSpec((B, 32), lambda b, idx: (0, 0)),
            pl.BlockSpec((B, 8), lambda b, idx: (0, 0)),
            pl.BlockSpec((H, H), lambda b, idx: (0, 0)),
            pl.BlockSpec((32, H), lambda b, idx: (0, 0)),
            pl.BlockSpec((H,), lambda b, idx: (0,)),
            pl.BlockSpec((8, H), lambda b, idx: (0, 0)),
            pl.BlockSpec((H,), lambda b, idx: (0,)),
            pl.BlockSpec((H, 12), lambda b, idx: (0, 0)),
            pl.BlockSpec((12,), lambda b, idx: (0,)),
        ],
        out_specs=pl.BlockSpec((1, T, 12), lambda b, idx: (b, 0, 0)),
    )
    return pl.pallas_call(
        _d_body,
        grid_spec=grid_spec,
        out_shape=jax.ShapeDtypeStruct((B, T, 12), jnp.float32),
    )(batch_idx, proj_final, rc, rd, rp['Wc'][:H], rp['Wc'][H:], rp['bc'],
      rp['Wd'], rp['bd'], wh, bh)


def kernel(era5, basinContinuous, basinDiscrete, riverContinuous,
           riverDiscrete, edge_index, nodes, params):
    src = edge_index[0]
    dst = edge_index[1]
    src_tbl = jnp.pad(src.reshape(NTILES, PER_TILE),
                      ((0, 0), (0, CAP - PER_TILE)))
    dst_tbl = jnp.pad(dst.reshape(NTILES, PER_TILE),
                      ((0, 0), (0, CAP - PER_TILE)),
                      constant_values=N).reshape(NTILES, NCHUNK, CHUNK)
    zeros128 = jnp.zeros((128, H), jnp.float32)
    zeros640 = jnp.zeros((RPT, H), jnp.float32)
    ones128 = jnp.ones((CHUNK, H), jnp.float32)
    era5_t = jnp.transpose(era5, (1, 0, 2))

    degp3 = _sc_deg(dst_tbl, ones128, zeros128)

    blocks = params['blocks']
    proj, hp, dinv = _proj_kernel(era5_t, basinContinuous, basinDiscrete,
                                  degp3, params['basinProj'], blocks[0])

    Pa = _sc_agg(hp.reshape(T * N, H), src_tbl, dst_tbl, zeros640)
    hp2 = _b_kernel(Pa, hp, dinv, blocks[0]['gcn_W2'], blocks[0]['gcn_b2'])
    Pb = _sc_agg(hp2.reshape(T * N, H), src_tbl, dst_tbl, zeros640)
    proj1, hp_b1 = _c_kernel_mid(Pb, hp2, dinv, proj, blocks[0], blocks[1])

    Pc = _sc_agg(hp_b1.reshape(T * N, H), src_tbl, dst_tbl, zeros640)
    hp2_b1 = _b_kernel(Pc, hp_b1, dinv,
                       blocks[1]['gcn_W2'], blocks[1]['gcn_b2'])
    Pd = _sc_agg(hp2_b1.reshape(T * N, H), src_tbl, dst_tbl, zeros640)
    proj_final, hh, cc = _c_kernel_final(Pd, hp2_b1, dinv, proj1, blocks[1])

    batch_idx = _idx_kernel(nodes).reshape(B)
    cast = _d_kernel(batch_idx, proj_final, riverContinuous, riverDiscrete,
                     params['riverProj'], params['head'])
    return cast, hh, cc
